# baseline (speedup 1.0000x reference)
import functools

import jax
import jax.numpy as jnp
from jax.experimental import pallas as pl
from jax.experimental.pallas import tpu as pltpu

_PAD = 8


def _linear_kernel(z_ref, w_ref, b_ref, o_ref):
    acc = jnp.dot(z_ref[...], w_ref[...], preferred_element_type=jnp.float32)
    o_ref[...] = (acc + b_ref[...]).astype(o_ref.dtype)


def _linear(z, w, b, *, tn=8192):
    B, K = z.shape
    N = w.shape[1]
    tn = min(tn, N)
    return pl.pallas_call(
        _linear_kernel,
        out_shape=jax.ShapeDtypeStruct((B, N), jnp.bfloat16),
        grid=(N // tn,),
        in_specs=[
            pl.BlockSpec((B, K), lambda j: (0, 0)),
            pl.BlockSpec((K, tn), lambda j: (0, j)),
            pl.BlockSpec((1, tn), lambda j: (0, j)),
        ],
        out_specs=pl.BlockSpec((B, tn), lambda j: (0, j)),
        compiler_params=pltpu.CompilerParams(dimension_semantics=("parallel",)),
    )(z.astype(jnp.bfloat16), w.astype(jnp.bfloat16), b.reshape(1, N))


def _conv_kernel(x_ref, s_ref, t_ref, w_ref, b_ref, *rest,
                 H, W, up, pre_act, slope, act, stats):
    if stats:
        o_ref, sum_ref, sq_ref, pad_ref = rest
    else:
        o_ref, pad_ref = rest
    v = x_ref[0].astype(jnp.float32)
    if pre_act:
        v = v * s_ref[...] + t_ref[...]
        v = jnp.where(v >= 0.0, v, slope * v)
    vb = v.astype(jnp.bfloat16)
    if up:
        vb = jnp.repeat(jnp.repeat(vb, 2, axis=0), 2, axis=1)

    pad_ref[...] = jnp.zeros_like(pad_ref)
    pad_ref[1:H + 1, _PAD:_PAD + W, :] = vb

    acc = None
    for dh in range(3):
        for dw in range(3):
            patch = pad_ref[dh:dh + H, _PAD - 1 + dw:_PAD - 1 + dw + W, :]
            d = jax.lax.dot_general(
                patch, w_ref[dh, dw],
                dimension_numbers=(((2,), (0,)), ((), ())),
                preferred_element_type=jnp.float32,
            )
            acc = d if acc is None else acc + d
    y = acc + b_ref[...]
    if act == "tanh":
        y = jnp.tanh(y)
    o_ref[0] = y.astype(o_ref.dtype)

    if stats:
        sum_ref[0] = jnp.sum(jnp.sum(y, axis=0), axis=0, keepdims=True)
        sq_ref[0] = jnp.sum(jnp.sum(y * y, axis=0), axis=0, keepdims=True)


def _conv(x, w, b, *, up, scale=None, shift=None, slope=0.2, act="none",
          out_dtype=jnp.bfloat16, stats=True):
    B, h, ww, Cin = x.shape
    H = 2 * h if up else h
    W = 2 * ww if up else ww
    Cout = w.shape[-1]
    pre_act = scale is not None
    if scale is None:
        scale = jnp.ones((Cin,), jnp.float32)
        shift = jnp.zeros((Cin,), jnp.float32)
    kern = functools.partial(_conv_kernel, H=H, W=W, up=up,
                             pre_act=pre_act, slope=slope, act=act, stats=stats)
    out_shape = [jax.ShapeDtypeStruct((B, H, W, Cout), out_dtype)]
    out_specs = [pl.BlockSpec((1, H, W, Cout), lambda i: (i, 0, 0, 0))]
    if stats:
        out_shape += [jax.ShapeDtypeStruct((B, 1, Cout), jnp.float32)] * 2
        out_specs += [pl.BlockSpec((1, 1, Cout), lambda i: (i, 0, 0))] * 2
    res = pl.pallas_call(
        kern,
        out_shape=tuple(out_shape),
        grid_spec=pltpu.PrefetchScalarGridSpec(
            num_scalar_prefetch=0,
            grid=(B,),
            in_specs=[
                pl.BlockSpec((1, h, ww, Cin), lambda i: (i, 0, 0, 0)),
                pl.BlockSpec((1, Cin), lambda i: (0, 0)),
                pl.BlockSpec((1, Cin), lambda i: (0, 0)),
                pl.BlockSpec((3, 3, Cin, Cout), lambda i: (0, 0, 0, 0)),
                pl.BlockSpec((1, Cout), lambda i: (0, 0)),
            ],
            out_specs=out_specs,
            scratch_shapes=[
                pltpu.VMEM((H + 2, W + 2 * _PAD, Cin), jnp.bfloat16)],
        ),
        compiler_params=pltpu.CompilerParams(
            dimension_semantics=("parallel",),
            vmem_limit_bytes=64 * 1024 * 1024,
        ),
    )(x, scale.reshape(1, Cin), shift.reshape(1, Cin),
      w.astype(jnp.bfloat16), b.reshape(1, Cout))
    return res if stats else res[0]


def _bn_scale_shift(ssum, ssq, gamma, beta, count, eps=0.8):
    mean = jnp.sum(ssum, axis=(0, 1)) / count
    var = jnp.sum(ssq, axis=(0, 1)) / count - mean * mean
    scale = gamma * jax.lax.rsqrt(var + eps)
    shift = beta - mean * scale
    return scale, shift


def kernel(z, l1_w, l1_b, c1_w, c1_b, bn1_g, bn1_b,
           c2_w, c2_b, bn2_g, bn2_b, c3_w, c3_b):
    B = z.shape[0]
    init = 16

    h = _linear(z, l1_w, l1_b)
    x = h.reshape(B, init, init, 128)

    c1, s1, q1 = _conv(x, c1_w, c1_b, up=True)
    sc1, sh1 = _bn_scale_shift(s1, q1, bn1_g, bn1_b, B * 32 * 32)

    c2, s2, q2 = _conv(c1, c2_w, c2_b, up=True, scale=sc1, shift=sh1)
    sc2, sh2 = _bn_scale_shift(s2, q2, bn2_g, bn2_b, B * 64 * 64)

    c3 = _conv(c2, c3_w, c3_b, up=False, scale=sc2, shift=sh2,
               act="tanh", out_dtype=jnp.float32, stats=False)
    return c3.transpose(0, 3, 1, 2)
